# baseline (speedup 1.0000x reference)
import functools

import jax
import jax.numpy as jnp
from jax import lax
from jax.experimental import pallas as pl
from jax.experimental.pallas import tpu as pltpu
from jax.experimental.pallas import tpu_sc as plsc

C = 19
S = 512
HW = S * S
NIMG = 4
NPIX = NIMG * HW
NW = 32
BW = 256
BPX = 8 * BW
NCHUNK = (HW // NW) // BPX
NB = 2048
NBC = C * NB


def _tree_sum(vals):
    while len(vals) > 1:
        nxt = [a + b for a, b in zip(vals[::2], vals[1::2])]
        if len(vals) % 2:
            nxt.append(vals[-1])
        vals = nxt
    return vals[0]


def _sc_hist_body(pred_hbm, tgt_hbm, out_c_hbm,
                  pbuf_a, pbuf_b, tbuf_a, tbuf_b, hist_c,
                  sem_a, sem_b):
    nc = 2
    wid = lax.axis_index("s") * nc + lax.axis_index("c")
    img = lax.shift_right_logical(wid, 3)
    sub = lax.bitwise_and(wid, 7)
    row0 = sub * 64

    zi = jnp.zeros((16,), jnp.int32)

    def zero_body(i, carry):
        hist_c[pl.ds(i * 16, 16)] = zi
        return carry

    def issue(j, pbuf, tbuf, sem):
        r0 = row0 + lax.shift_right_logical(j, 1) * 8
        c0 = lax.bitwise_and(j, 1) * BW
        pltpu.async_copy(
            tgt_hbm.at[img, pl.ds(r0, 8), pl.ds(c0, BW)], tbuf, sem)
        for c in range(C):
            pltpu.async_copy(
                pred_hbm.at[img * C + c, pl.ds(r0, 8), pl.ds(c0, BW)],
                pbuf.at[c], sem)

    def drain(pbuf, tbuf, sem):
        pltpu.make_async_copy(
            tgt_hbm.at[0, pl.ds(0, 8), pl.ds(0, BW)], tbuf, sem).wait()
        pltpu.make_async_copy(
            pred_hbm.at[pl.ds(0, C), pl.ds(0, 8), pl.ds(0, BW)],
            pbuf, sem).wait()

    def compute(pbuf, tbuf):
        zi16 = jnp.zeros((16,), jnp.int32)
        ones_i = jnp.full((16,), 1, jnp.int32)
        neg1_i = jnp.full((16,), -1, jnp.int32)
        fgfix_i = jnp.full((16,), 65537, jnp.int32)

        @plsc.parallel_loop(0, BW // 16, unroll=4)
        def col_body(cb):
            col0 = cb * 16
            for r in range(8):
                lbl = tbuf[r, pl.ds(col0, 16)]
                ys = [jnp.exp(pbuf[c, r, pl.ds(col0, 16)]) for c in range(C)]
                r_nb = float(NB) / _tree_sum(list(ys))
                idx_bgf = zi16
                for c in range(C):
                    q = (ys[c] * r_nb).astype(jnp.int32)
                    idx = jnp.maximum((c * NB + NB - 1) - q, c * NB)
                    plsc.addupdate_scatter(hist_c, [idx], ones_i)
                    idx_bgf = jnp.where(lbl == c, idx, idx_bgf)
                lbl_nb2 = lbl * (2 * NB)
                idx_fg = (lbl_nb2 + (NB - 1)) - idx_bgf
                plsc.addupdate_scatter(hist_c, [idx_bgf], neg1_i)
                plsc.addupdate_scatter(hist_c, [idx_fg], fgfix_i)

    issue(0, pbuf_a, tbuf_a, sem_a)
    issue(1, pbuf_b, tbuf_b, sem_b)
    lax.fori_loop(0, NBC // 16, zero_body, 0)

    def outer_body(jj, carry):
        j = jj * 2
        drain(pbuf_a, tbuf_a, sem_a)
        compute(pbuf_a, tbuf_a)

        @pl.when(jj < NCHUNK // 2 - 1)
        def _():
            issue(j + 2, pbuf_a, tbuf_a, sem_a)

        drain(pbuf_b, tbuf_b, sem_b)
        compute(pbuf_b, tbuf_b)

        @pl.when(jj < NCHUNK // 2 - 1)
        def _():
            issue(j + 3, pbuf_b, tbuf_b, sem_b)

        return carry

    lax.fori_loop(0, NCHUNK // 2, outer_body, 0)

    pltpu.sync_copy(hist_c, out_c_hbm.at[pl.ds(wid * NBC, NBC)])


def _cumsum_lanes(x):
    lanes = x.shape[-1]
    lane_idx = lax.broadcasted_iota(jnp.int32, x.shape, x.ndim - 1)
    sh = 1
    while sh < lanes:
        rolled = pltpu.roll(x, sh, axis=x.ndim - 1)
        x = x + jnp.where(lane_idx >= sh, rolled, 0.0)
        sh *= 2
    return x


def _tc_finish_body(hc_ref, out_ref):
    hcu = lax.bitcast_convert_type(hc_ref[...], jnp.uint32)
    n_t = (hcu & jnp.uint32(0xFFFF)).astype(jnp.float32)
    g_t = (hcu >> jnp.uint32(16)).astype(jnp.float32)
    col_cls = lax.broadcasted_iota(jnp.int32, (C, NW * C), 1) % C
    row_cls = lax.broadcasted_iota(jnp.int32, (C, NW * C), 0)
    sel = (col_cls == row_cls).astype(jnp.float32)
    n = jnp.dot(sel, n_t, preferred_element_type=jnp.float32)
    g = jnp.dot(sel, g_t, preferred_element_type=jnp.float32)
    F = _cumsum_lanes(g)
    B = _cumsum_lanes(n - g)
    G = jnp.sum(g, axis=1, keepdims=True)
    denom = G + B
    J = jnp.where(denom > 0, 1.0 - (G - F) / jnp.maximum(denom, 1.0), 0.0)
    lane_idx = lax.broadcasted_iota(jnp.int32, J.shape, 1)
    j_prev = jnp.where(lane_idx == 0, 0.0, pltpu.roll(J, 1, axis=1))
    d_j = J - j_prev
    mid = (float(NB) - 0.5 - lane_idx.astype(jnp.float32)) * (1.0 / NB)
    out_ref[...] = jnp.reshape(jnp.sum(mid * d_j) / float(C), (1, 1))


_sc_hist = functools.partial(
    pl.kernel,
    out_type=jax.ShapeDtypeStruct((NW * NBC,), jnp.int32),
    scratch_types=[
        pltpu.VMEM((C, 8, BW), jnp.float32),
        pltpu.VMEM((C, 8, BW), jnp.float32),
        pltpu.VMEM((8, BW), jnp.int32),
        pltpu.VMEM((8, BW), jnp.int32),
        pltpu.VMEM((NBC,), jnp.int32),
        pltpu.SemaphoreType.DMA,
        pltpu.SemaphoreType.DMA,
    ],
    mesh=plsc.VectorSubcoreMesh(core_axis_name="c", subcore_axis_name="s"),
    compiler_params=pltpu.CompilerParams(needs_layout_passes=False),
)(_sc_hist_body)


def kernel(prediction, target):
    pred3 = prediction.reshape(NIMG * C, S, S)
    hc = _sc_hist(pred3, target)
    loss = pl.pallas_call(
        _tc_finish_body,
        out_shape=jax.ShapeDtypeStruct((1, 1), jnp.float32),
    )(hc.reshape(NW * C, NB))
    return loss[0, 0]


# speedup vs baseline: 1.1219x; 1.1219x over previous
import functools

import jax
import jax.numpy as jnp
from jax import lax
from jax.experimental import pallas as pl
from jax.experimental.pallas import tpu as pltpu
from jax.experimental.pallas import tpu_sc as plsc

C = 19
S = 512
HW = S * S
NIMG = 4
NPIX = NIMG * HW
NW = 32
BW = 256
BPX = 8 * BW
NCHUNK = (HW // NW) // BPX
NB = 2048
NBC = C * NB


def _tree_sum(vals):
    while len(vals) > 1:
        nxt = [a + b for a, b in zip(vals[::2], vals[1::2])]
        if len(vals) % 2:
            nxt.append(vals[-1])
        vals = nxt
    return vals[0]


def _sc_hist_body(pred_hbm, tgt_hbm, out_c_hbm,
                  pbuf_a, pbuf_b, tbuf_a, tbuf_b, hist_c,
                  sem_a, sem_b):
    nc = 2
    wid = lax.axis_index("s") * nc + lax.axis_index("c")
    img = lax.shift_right_logical(wid, 3)
    sub = lax.bitwise_and(wid, 7)
    row0 = sub * 64

    zi = jnp.zeros((16,), jnp.int32)

    def zero_body(i, carry):
        hist_c[pl.ds(i * 16, 16)] = zi
        return carry

    def issue(j, pbuf, tbuf, sem):
        r0 = row0 + lax.shift_right_logical(j, 1) * 8
        c0 = lax.bitwise_and(j, 1) * BW
        pltpu.async_copy(
            tgt_hbm.at[img, pl.ds(r0, 8), pl.ds(c0, BW)], tbuf, sem)
        for c in range(C):
            pltpu.async_copy(
                pred_hbm.at[img * C + c, pl.ds(r0, 8), pl.ds(c0, BW)],
                pbuf.at[c], sem)

    def drain(pbuf, tbuf, sem):
        pltpu.make_async_copy(
            tgt_hbm.at[0, pl.ds(0, 8), pl.ds(0, BW)], tbuf, sem).wait()
        pltpu.make_async_copy(
            pred_hbm.at[pl.ds(0, C), pl.ds(0, 8), pl.ds(0, BW)],
            pbuf, sem).wait()

    def compute(pbuf, tbuf):
        zi16 = jnp.zeros((16,), jnp.int32)
        ones_i = jnp.full((16,), 1, jnp.int32)
        neg1_i = jnp.full((16,), -1, jnp.int32)
        fgfix_i = jnp.full((16,), 65537, jnp.int32)

        @plsc.parallel_loop(0, BW // 16, unroll=2)
        def col_body(cb):
            col0 = cb * 16
            for r in range(8):
                lbl = tbuf[r, pl.ds(col0, 16)]
                ys = [jnp.exp(pbuf[c, r, pl.ds(col0, 16)]) for c in range(C)]
                r_nb = float(NB) / _tree_sum(list(ys))
                idx_bgf = zi16
                for c in range(C):
                    q = (ys[c] * r_nb).astype(jnp.int32)
                    idx = jnp.maximum((c * NB + NB - 1) - q, c * NB)
                    plsc.addupdate_scatter(hist_c, [idx], ones_i)
                    idx_bgf = jnp.where(lbl == c, idx, idx_bgf)
                lbl_nb2 = lbl * (2 * NB)
                idx_fg = (lbl_nb2 + (NB - 1)) - idx_bgf
                plsc.addupdate_scatter(hist_c, [idx_bgf], neg1_i)
                plsc.addupdate_scatter(hist_c, [idx_fg], fgfix_i)

    issue(0, pbuf_a, tbuf_a, sem_a)
    issue(1, pbuf_b, tbuf_b, sem_b)
    lax.fori_loop(0, NBC // 16, zero_body, 0)

    def outer_body(jj, carry):
        j = jj * 2
        drain(pbuf_a, tbuf_a, sem_a)
        compute(pbuf_a, tbuf_a)

        @pl.when(jj < NCHUNK // 2 - 1)
        def _():
            issue(j + 2, pbuf_a, tbuf_a, sem_a)

        drain(pbuf_b, tbuf_b, sem_b)
        compute(pbuf_b, tbuf_b)

        @pl.when(jj < NCHUNK // 2 - 1)
        def _():
            issue(j + 3, pbuf_b, tbuf_b, sem_b)

        return carry

    lax.fori_loop(0, NCHUNK // 2, outer_body, 0)

    pltpu.sync_copy(hist_c, out_c_hbm.at[pl.ds(wid * NBC, NBC)])


def _cumsum_lanes(x):
    lanes = x.shape[-1]
    lane_idx = lax.broadcasted_iota(jnp.int32, x.shape, x.ndim - 1)
    sh = 1
    while sh < lanes:
        rolled = pltpu.roll(x, sh, axis=x.ndim - 1)
        x = x + jnp.where(lane_idx >= sh, rolled, 0.0)
        sh *= 2
    return x


def _tc_finish_body(hc_ref, out_ref):
    hcu = lax.bitcast_convert_type(hc_ref[...], jnp.uint32)
    n_t = (hcu & jnp.uint32(0xFFFF)).astype(jnp.float32)
    g_t = (hcu >> jnp.uint32(16)).astype(jnp.float32)
    col_cls = lax.broadcasted_iota(jnp.int32, (C, NW * C), 1) % C
    row_cls = lax.broadcasted_iota(jnp.int32, (C, NW * C), 0)
    sel = (col_cls == row_cls).astype(jnp.float32)
    n = jnp.dot(sel, n_t, preferred_element_type=jnp.float32)
    g = jnp.dot(sel, g_t, preferred_element_type=jnp.float32)
    F = _cumsum_lanes(g)
    B = _cumsum_lanes(n - g)
    G = jnp.sum(g, axis=1, keepdims=True)
    denom = G + B
    J = jnp.where(denom > 0, 1.0 - (G - F) / jnp.maximum(denom, 1.0), 0.0)
    lane_idx = lax.broadcasted_iota(jnp.int32, J.shape, 1)
    j_prev = jnp.where(lane_idx == 0, 0.0, pltpu.roll(J, 1, axis=1))
    d_j = J - j_prev
    mid = (float(NB) - 0.5 - lane_idx.astype(jnp.float32)) * (1.0 / NB)
    out_ref[...] = jnp.reshape(jnp.sum(mid * d_j) / float(C), (1, 1))


_sc_hist = functools.partial(
    pl.kernel,
    out_type=jax.ShapeDtypeStruct((NW * NBC,), jnp.int32),
    scratch_types=[
        pltpu.VMEM((C, 8, BW), jnp.float32),
        pltpu.VMEM((C, 8, BW), jnp.float32),
        pltpu.VMEM((8, BW), jnp.int32),
        pltpu.VMEM((8, BW), jnp.int32),
        pltpu.VMEM((NBC,), jnp.int32),
        pltpu.SemaphoreType.DMA,
        pltpu.SemaphoreType.DMA,
    ],
    mesh=plsc.VectorSubcoreMesh(core_axis_name="c", subcore_axis_name="s"),
    compiler_params=pltpu.CompilerParams(needs_layout_passes=False),
)(_sc_hist_body)


def kernel(prediction, target):
    pred3 = prediction.reshape(NIMG * C, S, S)
    hc = _sc_hist(pred3, target)
    loss = pl.pallas_call(
        _tc_finish_body,
        out_shape=jax.ShapeDtypeStruct((1, 1), jnp.float32),
    )(hc.reshape(NW * C, NB))
    return loss[0, 0]
